# baseline (speedup 1.0000x reference)
import functools

import jax
import jax.numpy as jnp
from jax import lax
from jax.experimental import pallas as pl
from jax.experimental.pallas import tpu as pltpu

_K = 32
_N = 100000
_Q = 1024
_R = 8
_CB = 7
_BLK = 14336
_SUB = 3584
_RPB = 8
_NSUB = _BLK // _SUB
_NPAD = _CB * _BLK
_METABITS = 10
_META_END = _N << _METABITS
_BIGI = 1 << 30
_BIGL = 1 << 20


def _body(d_ref, m_ref, o_ref, vv_ref, vm_ref):
    c = pl.program_id(1)

    @pl.when(c == 0)
    def _init():
        vv_ref[...] = jnp.full((_R, _K), jnp.inf, dtype=jnp.float32)
        vm_ref[...] = -(lax.broadcasted_iota(jnp.int32, (_R, _K), 1) + 1)

    def subchunk(j, carry):
        vv, vm = carry
        off = pl.multiple_of(j * _SUB, _SUB)
        ch = d_ref[:, pl.ds(off, _SUB)]
        mb = jnp.broadcast_to(m_ref[0, :, pl.ds(off, _SUB)], (_R, _SUB))
        ch = jnp.where(mb >= _META_END, jnp.inf, ch)

        def cond(st):
            ch_, vv_, _ = st
            t = jnp.max(vv_, axis=1, keepdims=True)
            return jnp.any(ch_ < t)

        def one_round(st):
            ch_, vv_, vm_ = st
            t = jnp.max(vv_, axis=1, keepdims=True)
            rowmin = jnp.min(ch_, axis=1, keepdims=True)
            admit = rowmin < t
            eqm = ch_ == rowmin
            metamin = jnp.min(jnp.where(eqm, mb, _BIGI), axis=1, keepdims=True)
            evsel = vv_ == t
            evmeta = jnp.max(jnp.where(evsel, vm_, -_BIGI), axis=1, keepdims=True)
            upd = evsel & (vm_ == evmeta) & admit
            vv_ = jnp.where(upd, rowmin, vv_)
            vm_ = jnp.where(upd, metamin, vm_)
            ch_ = jnp.where(eqm & (mb == metamin) & admit, jnp.inf, ch_)
            return ch_, vv_, vm_

        def round_batch(st):
            for _ in range(_RPB):
                st = one_round(st)
            return st

        _, vv, vm = lax.while_loop(cond, round_batch, (ch, vv, vm))
        return vv, vm

    vv, vm = lax.fori_loop(0, _NSUB, subchunk, (vv_ref[...], vm_ref[...]))
    vv_ref[...] = vv
    vm_ref[...] = vm

    @pl.when(c == _CB - 1)
    def _vote():
        labels = vm_ref[...] & ((1 << _METABITS) - 1)
        counts = jnp.zeros((_R, _K), dtype=jnp.int32)
        for jj in range(_K):
            counts = counts + (labels == labels[:, jj:jj + 1]).astype(jnp.int32)
        maxc = jnp.max(counts, axis=1, keepdims=True)
        winner = jnp.min(jnp.where(counts == maxc, labels, _BIGL), axis=1)
        o_ref[...] = winner.reshape(1, 1, _R)


@jax.jit
def kernel(distances, labels):
    meta = (jnp.arange(_N, dtype=jnp.int32) << _METABITS) | labels.astype(jnp.int32)
    meta = jnp.pad(meta, (0, _NPAD - _N), constant_values=_META_END)
    meta3 = meta.reshape(_CB, 1, _BLK)
    out = pl.pallas_call(
        _body,
        grid=(_Q // _R, _CB),
        in_specs=[
            pl.BlockSpec((_R, _BLK), lambda i, c: (i, c)),
            pl.BlockSpec((1, 1, _BLK), lambda i, c: (c, 0, 0)),
        ],
        out_specs=pl.BlockSpec((1, 1, _R), lambda i, c: (i, 0, 0)),
        out_shape=jax.ShapeDtypeStruct((_Q // _R, 1, _R), jnp.int32),
        scratch_shapes=[
            pltpu.VMEM((_R, _K), jnp.float32),
            pltpu.VMEM((_R, _K), jnp.int32),
        ],
        compiler_params=pltpu.CompilerParams(
            dimension_semantics=("arbitrary", "arbitrary"),
        ),
    )(distances, meta3)
    return out.reshape(_Q)


# speedup vs baseline: 1.5814x; 1.5814x over previous
import functools

import jax
import jax.numpy as jnp
from jax import lax
from jax.experimental import pallas as pl
from jax.experimental.pallas import tpu as pltpu

_K = 32
_N = 100000
_Q = 1024
_R = 16
_CB = 7
_BLK = 14336
_SUB = 7168
_RPB = 1
_NSUB = _BLK // _SUB
_NPAD = _CB * _BLK
_METABITS = 10
_META_END = _N << _METABITS
_BIGI = 1 << 30
_BIGL = 1 << 20


def _body(d_ref, m_ref, o_ref, vv_ref, vm_ref):
    c = pl.program_id(1)

    @pl.when(c == 0)
    def _init():
        vv_ref[...] = jnp.full((_R, _K), jnp.inf, dtype=jnp.float32)
        vm_ref[...] = -(lax.broadcasted_iota(jnp.int32, (_R, _K), 1) + 1)

    def subchunk(j, carry):
        vv, vm = carry
        off = pl.multiple_of(j * _SUB, _SUB)
        ch = d_ref[:, pl.ds(off, _SUB)]
        mb = jnp.broadcast_to(m_ref[0, :, pl.ds(off, _SUB)], (_R, _SUB))
        ch = jnp.where(mb >= _META_END, jnp.inf, ch)

        def cond(st):
            ch_, vv_, _ = st
            t = jnp.max(vv_, axis=1, keepdims=True)
            return jnp.any(ch_ < t)

        def one_round(st):
            ch_, vv_, vm_ = st
            t = jnp.max(vv_, axis=1, keepdims=True)
            rowmin = jnp.min(ch_, axis=1, keepdims=True)
            admit = rowmin < t
            eqm = ch_ == rowmin
            metamin = jnp.min(jnp.where(eqm, mb, _BIGI), axis=1, keepdims=True)
            evsel = vv_ == t
            evmeta = jnp.max(jnp.where(evsel, vm_, -_BIGI), axis=1, keepdims=True)
            upd = evsel & (vm_ == evmeta) & admit
            vv_ = jnp.where(upd, rowmin, vv_)
            vm_ = jnp.where(upd, metamin, vm_)
            ch_ = jnp.where(eqm & (mb == metamin) & admit, jnp.inf, ch_)
            return ch_, vv_, vm_

        def round_batch(st):
            for _ in range(_RPB):
                st = one_round(st)
            return st

        _, vv, vm = lax.while_loop(cond, round_batch, (ch, vv, vm))
        return vv, vm

    vv, vm = lax.fori_loop(0, _NSUB, subchunk, (vv_ref[...], vm_ref[...]))
    vv_ref[...] = vv
    vm_ref[...] = vm

    @pl.when(c == _CB - 1)
    def _vote():
        labels = vm_ref[...] & ((1 << _METABITS) - 1)
        counts = jnp.zeros((_R, _K), dtype=jnp.int32)
        for jj in range(_K):
            counts = counts + (labels == labels[:, jj:jj + 1]).astype(jnp.int32)
        maxc = jnp.max(counts, axis=1, keepdims=True)
        winner = jnp.min(jnp.where(counts == maxc, labels, _BIGL), axis=1)
        o_ref[...] = winner.reshape(1, 1, _R)


@jax.jit
def kernel(distances, labels):
    meta = (jnp.arange(_N, dtype=jnp.int32) << _METABITS) | labels.astype(jnp.int32)
    meta = jnp.pad(meta, (0, _NPAD - _N), constant_values=_META_END)
    meta3 = meta.reshape(_CB, 1, _BLK)
    out = pl.pallas_call(
        _body,
        grid=(_Q // _R, _CB),
        in_specs=[
            pl.BlockSpec((_R, _BLK), lambda i, c: (i, c)),
            pl.BlockSpec((1, 1, _BLK), lambda i, c: (c, 0, 0)),
        ],
        out_specs=pl.BlockSpec((1, 1, _R), lambda i, c: (i, 0, 0)),
        out_shape=jax.ShapeDtypeStruct((_Q // _R, 1, _R), jnp.int32),
        scratch_shapes=[
            pltpu.VMEM((_R, _K), jnp.float32),
            pltpu.VMEM((_R, _K), jnp.int32),
        ],
        compiler_params=pltpu.CompilerParams(
            dimension_semantics=("arbitrary", "arbitrary"),
        ),
    )(distances, meta3)
    return out.reshape(_Q)


# speedup vs baseline: 2.0373x; 1.2883x over previous
import functools

import jax
import jax.numpy as jnp
from jax import lax
from jax.experimental import pallas as pl
from jax.experimental.pallas import tpu as pltpu

_K = 32
_N = 100000
_Q = 1024
_R = 32
_CB = 7
_BLK = 14336
_SUB = 7168
_RPB = 1
_NSUB = _BLK // _SUB
_NPAD = _CB * _BLK
_METABITS = 10
_META_END = _N << _METABITS
_BIGI = 1 << 30
_BIGL = 1 << 20


def _body(d_ref, m_ref, o_ref, vv_ref, vm_ref):
    c = pl.program_id(1)

    @pl.when(c == 0)
    def _init():
        vv_ref[...] = jnp.full((_R, _K), jnp.inf, dtype=jnp.float32)
        vm_ref[...] = -(lax.broadcasted_iota(jnp.int32, (_R, _K), 1) + 1)

    def subchunk(j, carry):
        vv, vm = carry
        off = pl.multiple_of(j * _SUB, _SUB)
        ch = d_ref[:, pl.ds(off, _SUB)]
        mb = jnp.broadcast_to(m_ref[0, :, pl.ds(off, _SUB)], (_R, _SUB))
        ch = jnp.where(mb >= _META_END, jnp.inf, ch)

        def cond(st):
            ch_, vv_, _ = st
            t = jnp.max(vv_, axis=1, keepdims=True)
            return jnp.any(ch_ < t)

        def one_round(st):
            ch_, vv_, vm_ = st
            t = jnp.max(vv_, axis=1, keepdims=True)
            rowmin = jnp.min(ch_, axis=1, keepdims=True)
            admit = rowmin < t
            eqm = ch_ == rowmin
            metamin = jnp.min(jnp.where(eqm, mb, _BIGI), axis=1, keepdims=True)
            evsel = vv_ == t
            evmeta = jnp.max(jnp.where(evsel, vm_, -_BIGI), axis=1, keepdims=True)
            upd = evsel & (vm_ == evmeta) & admit
            vv_ = jnp.where(upd, rowmin, vv_)
            vm_ = jnp.where(upd, metamin, vm_)
            ch_ = jnp.where(eqm & (mb == metamin) & admit, jnp.inf, ch_)
            return ch_, vv_, vm_

        def round_batch(st):
            for _ in range(_RPB):
                st = one_round(st)
            return st

        _, vv, vm = lax.while_loop(cond, round_batch, (ch, vv, vm))
        return vv, vm

    vv, vm = lax.fori_loop(0, _NSUB, subchunk, (vv_ref[...], vm_ref[...]))
    vv_ref[...] = vv
    vm_ref[...] = vm

    @pl.when(c == _CB - 1)
    def _vote():
        labels = vm_ref[...] & ((1 << _METABITS) - 1)
        counts = jnp.zeros((_R, _K), dtype=jnp.int32)
        for jj in range(_K):
            counts = counts + (labels == labels[:, jj:jj + 1]).astype(jnp.int32)
        maxc = jnp.max(counts, axis=1, keepdims=True)
        winner = jnp.min(jnp.where(counts == maxc, labels, _BIGL), axis=1)
        o_ref[...] = winner.reshape(1, 1, _R)


@jax.jit
def kernel(distances, labels):
    meta = (jnp.arange(_N, dtype=jnp.int32) << _METABITS) | labels.astype(jnp.int32)
    meta = jnp.pad(meta, (0, _NPAD - _N), constant_values=_META_END)
    meta3 = meta.reshape(_CB, 1, _BLK)
    out = pl.pallas_call(
        _body,
        grid=(_Q // _R, _CB),
        in_specs=[
            pl.BlockSpec((_R, _BLK), lambda i, c: (i, c)),
            pl.BlockSpec((1, 1, _BLK), lambda i, c: (c, 0, 0)),
        ],
        out_specs=pl.BlockSpec((1, 1, _R), lambda i, c: (i, 0, 0)),
        out_shape=jax.ShapeDtypeStruct((_Q // _R, 1, _R), jnp.int32),
        scratch_shapes=[
            pltpu.VMEM((_R, _K), jnp.float32),
            pltpu.VMEM((_R, _K), jnp.int32),
        ],
        compiler_params=pltpu.CompilerParams(
            dimension_semantics=("arbitrary", "arbitrary"),
        ),
    )(distances, meta3)
    return out.reshape(_Q)


# speedup vs baseline: 2.3292x; 1.1432x over previous
import functools

import jax
import jax.numpy as jnp
from jax import lax
from jax.experimental import pallas as pl
from jax.experimental.pallas import tpu as pltpu

_K = 32
_N = 100000
_Q = 1024
_R = 64
_CB = 7
_BLK = 14336
_SUB = 7168
_RPB = 1
_NSUB = _BLK // _SUB
_NPAD = _CB * _BLK
_METABITS = 10
_META_END = _N << _METABITS
_BIGI = 1 << 30
_BIGL = 1 << 20


def _body(d_ref, m_ref, o_ref, vv_ref, vm_ref):
    c = pl.program_id(1)

    @pl.when(c == 0)
    def _init():
        vv_ref[...] = jnp.full((_R, _K), jnp.inf, dtype=jnp.float32)
        vm_ref[...] = -(lax.broadcasted_iota(jnp.int32, (_R, _K), 1) + 1)

    def subchunk(j, carry):
        vv, vm = carry
        off = pl.multiple_of(j * _SUB, _SUB)
        ch = d_ref[:, pl.ds(off, _SUB)]
        mb = jnp.broadcast_to(m_ref[0, :, pl.ds(off, _SUB)], (_R, _SUB))
        ch = jnp.where(mb >= _META_END, jnp.inf, ch)

        def cond(st):
            ch_, vv_, _ = st
            t = jnp.max(vv_, axis=1, keepdims=True)
            return jnp.any(ch_ < t)

        def one_round(st):
            ch_, vv_, vm_ = st
            t = jnp.max(vv_, axis=1, keepdims=True)
            rowmin = jnp.min(ch_, axis=1, keepdims=True)
            admit = rowmin < t
            eqm = ch_ == rowmin
            metamin = jnp.min(jnp.where(eqm, mb, _BIGI), axis=1, keepdims=True)
            evsel = vv_ == t
            evmeta = jnp.max(jnp.where(evsel, vm_, -_BIGI), axis=1, keepdims=True)
            upd = evsel & (vm_ == evmeta) & admit
            vv_ = jnp.where(upd, rowmin, vv_)
            vm_ = jnp.where(upd, metamin, vm_)
            ch_ = jnp.where(eqm & (mb == metamin) & admit, jnp.inf, ch_)
            return ch_, vv_, vm_

        def round_batch(st):
            for _ in range(_RPB):
                st = one_round(st)
            return st

        _, vv, vm = lax.while_loop(cond, round_batch, (ch, vv, vm))
        return vv, vm

    vv, vm = lax.fori_loop(0, _NSUB, subchunk, (vv_ref[...], vm_ref[...]))
    vv_ref[...] = vv
    vm_ref[...] = vm

    @pl.when(c == _CB - 1)
    def _vote():
        labels = vm_ref[...] & ((1 << _METABITS) - 1)
        counts = jnp.zeros((_R, _K), dtype=jnp.int32)
        for jj in range(_K):
            counts = counts + (labels == labels[:, jj:jj + 1]).astype(jnp.int32)
        maxc = jnp.max(counts, axis=1, keepdims=True)
        winner = jnp.min(jnp.where(counts == maxc, labels, _BIGL), axis=1)
        o_ref[...] = winner.reshape(1, 1, _R)


@jax.jit
def kernel(distances, labels):
    meta = (jnp.arange(_N, dtype=jnp.int32) << _METABITS) | labels.astype(jnp.int32)
    meta = jnp.pad(meta, (0, _NPAD - _N), constant_values=_META_END)
    meta3 = meta.reshape(_CB, 1, _BLK)
    out = pl.pallas_call(
        _body,
        grid=(_Q // _R, _CB),
        in_specs=[
            pl.BlockSpec((_R, _BLK), lambda i, c: (i, c)),
            pl.BlockSpec((1, 1, _BLK), lambda i, c: (c, 0, 0)),
        ],
        out_specs=pl.BlockSpec((1, 1, _R), lambda i, c: (i, 0, 0)),
        out_shape=jax.ShapeDtypeStruct((_Q // _R, 1, _R), jnp.int32),
        scratch_shapes=[
            pltpu.VMEM((_R, _K), jnp.float32),
            pltpu.VMEM((_R, _K), jnp.int32),
        ],
        compiler_params=pltpu.CompilerParams(
            dimension_semantics=("arbitrary", "arbitrary"),
        ),
    )(distances, meta3)
    return out.reshape(_Q)


# speedup vs baseline: 2.3798x; 1.0217x over previous
import functools

import jax
import jax.numpy as jnp
from jax import lax
from jax.experimental import pallas as pl
from jax.experimental.pallas import tpu as pltpu

_K = 32
_N = 100000
_Q = 1024
_R = 128
_CB = 7
_BLK = 14336
_SUB = 7168
_RPB = 1
_NSUB = _BLK // _SUB
_NPAD = _CB * _BLK
_METABITS = 10
_META_END = _N << _METABITS
_BIGI = 1 << 30
_BIGL = 1 << 20


def _body(d_ref, m_ref, o_ref, vv_ref, vm_ref):
    c = pl.program_id(1)

    @pl.when(c == 0)
    def _init():
        vv_ref[...] = jnp.full((_R, _K), jnp.inf, dtype=jnp.float32)
        vm_ref[...] = -(lax.broadcasted_iota(jnp.int32, (_R, _K), 1) + 1)

    def subchunk(j, carry):
        vv, vm = carry
        off = pl.multiple_of(j * _SUB, _SUB)
        ch = d_ref[:, pl.ds(off, _SUB)]
        mb = jnp.broadcast_to(m_ref[0, :, pl.ds(off, _SUB)], (_R, _SUB))
        ch = jnp.where(mb >= _META_END, jnp.inf, ch)

        def cond(st):
            ch_, vv_, _ = st
            t = jnp.max(vv_, axis=1, keepdims=True)
            return jnp.any(ch_ < t)

        def one_round(st):
            ch_, vv_, vm_ = st
            t = jnp.max(vv_, axis=1, keepdims=True)
            rowmin = jnp.min(ch_, axis=1, keepdims=True)
            admit = rowmin < t
            eqm = ch_ == rowmin
            metamin = jnp.min(jnp.where(eqm, mb, _BIGI), axis=1, keepdims=True)
            evsel = vv_ == t
            evmeta = jnp.max(jnp.where(evsel, vm_, -_BIGI), axis=1, keepdims=True)
            upd = evsel & (vm_ == evmeta) & admit
            vv_ = jnp.where(upd, rowmin, vv_)
            vm_ = jnp.where(upd, metamin, vm_)
            ch_ = jnp.where(eqm & (mb == metamin) & admit, jnp.inf, ch_)
            return ch_, vv_, vm_

        def round_batch(st):
            for _ in range(_RPB):
                st = one_round(st)
            return st

        _, vv, vm = lax.while_loop(cond, round_batch, (ch, vv, vm))
        return vv, vm

    vv, vm = lax.fori_loop(0, _NSUB, subchunk, (vv_ref[...], vm_ref[...]))
    vv_ref[...] = vv
    vm_ref[...] = vm

    @pl.when(c == _CB - 1)
    def _vote():
        labels = vm_ref[...] & ((1 << _METABITS) - 1)
        counts = jnp.zeros((_R, _K), dtype=jnp.int32)
        for jj in range(_K):
            counts = counts + (labels == labels[:, jj:jj + 1]).astype(jnp.int32)
        maxc = jnp.max(counts, axis=1, keepdims=True)
        winner = jnp.min(jnp.where(counts == maxc, labels, _BIGL), axis=1)
        o_ref[...] = winner.reshape(1, 1, _R)


@jax.jit
def kernel(distances, labels):
    meta = (jnp.arange(_N, dtype=jnp.int32) << _METABITS) | labels.astype(jnp.int32)
    meta = jnp.pad(meta, (0, _NPAD - _N), constant_values=_META_END)
    meta3 = meta.reshape(_CB, 1, _BLK)
    out = pl.pallas_call(
        _body,
        grid=(_Q // _R, _CB),
        in_specs=[
            pl.BlockSpec((_R, _BLK), lambda i, c: (i, c)),
            pl.BlockSpec((1, 1, _BLK), lambda i, c: (c, 0, 0)),
        ],
        out_specs=pl.BlockSpec((1, 1, _R), lambda i, c: (i, 0, 0)),
        out_shape=jax.ShapeDtypeStruct((_Q // _R, 1, _R), jnp.int32),
        scratch_shapes=[
            pltpu.VMEM((_R, _K), jnp.float32),
            pltpu.VMEM((_R, _K), jnp.int32),
        ],
        compiler_params=pltpu.CompilerParams(
            dimension_semantics=("arbitrary", "arbitrary"),
        ),
    )(distances, meta3)
    return out.reshape(_Q)
